# baseline (speedup 1.0000x reference)
import functools
import math

import jax
import jax.numpy as jnp
from jax import lax
from jax.experimental import pallas as pl
from jax.experimental.pallas import tpu as pltpu
from jax.experimental.pallas import tpu_sc as plsc

_CBLK = 16384


def _tc_pack_table(table):
    v, d = table.shape
    tt = jnp.transpose(table)
    grid = -(-v // _CBLK)
    half = _CBLK // 2

    def body(in_ref, out_ref):
        t = jnp.transpose(in_ref[...])
        out_ref[...] = jnp.concatenate([t[:half], t[half:]], axis=1)

    return pl.pallas_call(
        body,
        grid=(grid,),
        in_specs=[pl.BlockSpec((d, _CBLK), lambda j: (0, j))],
        out_specs=pl.BlockSpec((half, 2 * d), lambda j: (j, 0)),
        out_shape=jax.ShapeDtypeStruct((grid * half, 2 * d), jnp.float32),
    )(tt)


def _tc_retile_out(flat, n_l, n_b, d, scale):
    inv = flat.reshape(n_l, n_b // 128, d, 128)
    nbb = n_b // 128
    bbq = 8

    def body(in_ref, out_ref):
        x = in_ref[0]
        cc = lax.broadcasted_iota(jnp.int32, (d, 2 * d), 1)
        pp = lax.broadcasted_iota(jnp.int32, (d, 2 * d), 0)
        s0 = (cc == 2 * pp).astype(jnp.float32)
        s1 = (cc == 2 * pp + 1).astype(jnp.float32)
        dn = (((1,), (0,)), ((), ()))
        for kk in range(d // 8):
            yk = lax.dot_general(
                x[:, :, kk * 8 : kk * 8 + 8], s0, dn
            ) + lax.dot_general(
                x[:, :, d + kk * 8 : d + kk * 8 + 8], s1, dn
            )
            out_ref[0, kk] = yk * scale

    return pl.pallas_call(
        body,
        grid=(n_l, nbb // bbq),
        in_specs=[pl.BlockSpec((1, bbq, d, 128), lambda l, q: (l, q, 0, 0))],
        out_specs=pl.BlockSpec(
            (1, d // 8, bbq, 8, 128), lambda l, q: (l, 0, q, 0, 0)
        ),
        out_shape=jax.ShapeDtypeStruct((n_l, d // 8, nbb, 8, 128), jnp.float32),
    )(inv)


def _make_embed_kernel(n_total, d_model, n_per_w, chunk, num_cores):
    n_chunks = n_per_w // chunk
    n_outer = n_chunks // 2
    mesh = plsc.VectorSubcoreMesh(core_axis_name="c", subcore_axis_name="s")

    scratch = (
        [pltpu.VMEM((chunk,), jnp.int32) for _ in range(2)]
        + [pltpu.VMEM((chunk, d_model), jnp.float32) for _ in range(2)]
        + [pltpu.SemaphoreType.DMA for _ in range(4)]
    )

    @functools.partial(
        pl.kernel,
        mesh=mesh,
        out_type=jax.ShapeDtypeStruct((n_total, d_model), jnp.float32),
        compiler_params=pltpu.CompilerParams(use_tc_tiling_on_sc=False),
        scratch_types=scratch,
    )
    def k(idx_hbm, table_hbm, out_hbm, *rest):
        idxb = rest[0:2]
        rows = rest[2:4]
        isem = rest[4:6]
        gsem = rest[6:8]
        wid = lax.axis_index("s") * num_cores + lax.axis_index("c")
        base = wid * n_per_w

        def i_start(c, b):
            pltpu.make_async_copy(
                idx_hbm.at[pl.ds(base + c * chunk, chunk)], idxb[b], isem[b]
            ).start()

        def i_wait(c, b):
            pltpu.make_async_copy(
                idx_hbm.at[pl.ds(base + c * chunk, chunk)], idxb[b], isem[b]
            ).wait()

        def remap(b):
            for g in range(chunk // 16):
                sl = pl.ds(g * 16, 16)
                vv = idxb[b][sl]
                idxb[b][sl] = (
                    (vv & jnp.int32(-16384))
                    + ((vv & 8191) * 2)
                    + ((vv >> 13) & 1)
                )

        def g_start(b):
            pltpu.make_async_copy(table_hbm.at[idxb[b]], rows[b], gsem[b]).start()

        def g_wait(b):
            pltpu.make_async_copy(table_hbm.at[idxb[b]], rows[b], gsem[b]).wait()

        def o_sync(c, b):
            pltpu.sync_copy(rows[b], out_hbm.at[pl.ds(base + c * chunk, chunk)])

        i_start(0, 0)
        i_start(1, 1)
        i_wait(0, 0)
        remap(0)
        g_start(0)

        def outer(i2, _):
            for b in range(2):
                c = i2 * 2 + b
                g_wait(b)

                @pl.when(i2 < n_outer - 1)
                def _():
                    i_start(c + 2, b)
                    i_wait(c + 1, 1 - b)
                    remap(1 - b)
                    g_start(1 - b)

                if b == 0:

                    @pl.when(i2 == n_outer - 1)
                    def _():
                        i_wait(c + 1, 1 - b)
                        remap(1 - b)
                        g_start(1 - b)

                o_sync(c, b)
            return 0

        lax.fori_loop(0, n_outer, outer, 0)

    return k


def kernel(x, table):
    b, l = x.shape
    v, d = table.shape
    n_total = b * l
    idx_lmajor = jnp.transpose(x).reshape(n_total).astype(jnp.int32)
    packed = _tc_pack_table(table)
    tbl_rows = packed.reshape(packed.shape[0] * 2, d)
    info = plsc.get_sparse_core_info()
    nw = info.num_cores * info.num_subcores
    n_per_w = n_total // nw
    k = _make_embed_kernel(n_total, d, n_per_w, 256, info.num_cores)
    out_flat = k(idx_lmajor, tbl_rows)
    tiled = _tc_retile_out(
        out_flat.reshape(n_total * d), l, b, d, float(math.sqrt(d))
    )
    final = jnp.transpose(tiled, (2, 4, 0, 1, 3)).reshape(b, l, d)
    return final


# speedup vs baseline: 2.1715x; 2.1715x over previous
import functools
import math

import jax
import jax.numpy as jnp
from jax import lax
from jax.experimental import pallas as pl
from jax.experimental.pallas import tpu as pltpu
from jax.experimental.pallas import tpu_sc as plsc

_CBLK = 16384


def _tc_pack_table(table):
    v, d = table.shape
    tt = jnp.transpose(table)
    grid = -(-v // _CBLK)
    half = _CBLK // 2

    def body(in_ref, out_ref):
        t = jnp.transpose(in_ref[...])
        out_ref[...] = jnp.concatenate([t[:half], t[half:]], axis=1)

    return pl.pallas_call(
        body,
        grid=(grid,),
        in_specs=[pl.BlockSpec((d, _CBLK), lambda j: (0, j))],
        out_specs=pl.BlockSpec((half, 2 * d), lambda j: (j, 0)),
        out_shape=jax.ShapeDtypeStruct((grid * half, 2 * d), jnp.float32),
    )(tt)


def _tc_retile_out(flat, n_l, n_b, d, scale):
    inv = flat.reshape(n_l, n_b // 128, d, 128)
    nbb = n_b // 128
    bbq = 8

    def body(in_ref, out_ref):
        x = in_ref[0]
        cc = lax.broadcasted_iota(jnp.int32, (d, 2 * d), 1)
        pp = lax.broadcasted_iota(jnp.int32, (d, 2 * d), 0)
        s0 = (cc == 2 * pp).astype(jnp.float32)
        s1 = (cc == 2 * pp + 1).astype(jnp.float32)
        dn = (((1,), (0,)), ((), ()))
        y = lax.dot_general(x[:, :, :d], s0, dn) + lax.dot_general(
            x[:, :, d:], s1, dn
        )
        y = y * scale
        for kk in range(d // 8):
            out_ref[0, kk] = y[:, kk * 8 : kk * 8 + 8, :]

    return pl.pallas_call(
        body,
        grid=(n_l, nbb // bbq),
        in_specs=[pl.BlockSpec((1, bbq, d, 128), lambda l, q: (l, q, 0, 0))],
        out_specs=pl.BlockSpec(
            (1, d // 8, bbq, 8, 128), lambda l, q: (l, 0, q, 0, 0)
        ),
        out_shape=jax.ShapeDtypeStruct((n_l, d // 8, nbb, 8, 128), jnp.float32),
    )(inv)


def _make_embed_kernel(n_total, d_model, n_per_w, chunk, num_cores):
    n_chunks = n_per_w // chunk
    n_outer = n_chunks // 2
    mesh = plsc.VectorSubcoreMesh(core_axis_name="c", subcore_axis_name="s")

    scratch = (
        [pltpu.VMEM((chunk,), jnp.int32) for _ in range(2)]
        + [pltpu.VMEM((chunk, d_model), jnp.float32) for _ in range(2)]
        + [pltpu.SemaphoreType.DMA for _ in range(4)]
    )

    @functools.partial(
        pl.kernel,
        mesh=mesh,
        out_type=jax.ShapeDtypeStruct((n_total, d_model), jnp.float32),
        compiler_params=pltpu.CompilerParams(use_tc_tiling_on_sc=False),
        scratch_types=scratch,
    )
    def k(idx_hbm, table_hbm, out_hbm, *rest):
        idxb = rest[0:2]
        rows = rest[2:4]
        isem = rest[4:6]
        gsem = rest[6:8]
        wid = lax.axis_index("s") * num_cores + lax.axis_index("c")
        base = wid * n_per_w

        def i_start(c, b):
            pltpu.make_async_copy(
                idx_hbm.at[pl.ds(base + c * chunk, chunk)], idxb[b], isem[b]
            ).start()

        def i_wait(c, b):
            pltpu.make_async_copy(
                idx_hbm.at[pl.ds(base + c * chunk, chunk)], idxb[b], isem[b]
            ).wait()

        def remap(b):
            for g in range(chunk // 16):
                sl = pl.ds(g * 16, 16)
                vv = idxb[b][sl]
                idxb[b][sl] = (
                    (vv & jnp.int32(-16384))
                    + ((vv & 8191) * 2)
                    + ((vv >> 13) & 1)
                )

        def g_start(b):
            pltpu.make_async_copy(table_hbm.at[idxb[b]], rows[b], gsem[b]).start()

        def g_wait(b):
            pltpu.make_async_copy(table_hbm.at[idxb[b]], rows[b], gsem[b]).wait()

        def o_sync(c, b):
            pltpu.sync_copy(rows[b], out_hbm.at[pl.ds(base + c * chunk, chunk)])

        i_start(0, 0)
        i_start(1, 1)
        i_wait(0, 0)
        remap(0)
        g_start(0)

        def outer(i2, _):
            for b in range(2):
                c = i2 * 2 + b
                g_wait(b)

                @pl.when(i2 < n_outer - 1)
                def _():
                    i_start(c + 2, b)
                    i_wait(c + 1, 1 - b)
                    remap(1 - b)
                    g_start(1 - b)

                if b == 0:

                    @pl.when(i2 == n_outer - 1)
                    def _():
                        i_wait(c + 1, 1 - b)
                        remap(1 - b)
                        g_start(1 - b)

                o_sync(c, b)
            return 0

        lax.fori_loop(0, n_outer, outer, 0)

    return k


def kernel(x, table):
    b, l = x.shape
    v, d = table.shape
    n_total = b * l
    idx_lmajor = jnp.transpose(x).reshape(n_total).astype(jnp.int32)
    packed = _tc_pack_table(table)
    tbl_rows = packed.reshape(packed.shape[0] * 2, d)
    info = plsc.get_sparse_core_info()
    nw = info.num_cores * info.num_subcores
    n_per_w = n_total // nw
    k = _make_embed_kernel(n_total, d, n_per_w, 256, info.num_cores)
    out_flat = k(idx_lmajor, tbl_rows)
    tiled = _tc_retile_out(
        out_flat.reshape(n_total * d), l, b, d, float(math.sqrt(d))
    )
    final = jnp.transpose(tiled, (2, 4, 0, 1, 3)).reshape(b, l, d)
    return final


# speedup vs baseline: 2.2166x; 1.0207x over previous
import functools
import math

import jax
import jax.numpy as jnp
from jax import lax
from jax.experimental import pallas as pl
from jax.experimental.pallas import tpu as pltpu
from jax.experimental.pallas import tpu_sc as plsc

_CBLK = 16384


def _tc_pack_table(table):
    v, d = table.shape
    tt = jnp.transpose(table)
    grid = -(-v // _CBLK)
    half = _CBLK // 2

    def body(in_ref, out_ref):
        t = jnp.transpose(in_ref[...])
        out_ref[...] = jnp.concatenate([t[:half], t[half:]], axis=1)

    return pl.pallas_call(
        body,
        grid=(grid,),
        in_specs=[pl.BlockSpec((d, _CBLK), lambda j: (0, j))],
        out_specs=pl.BlockSpec((half, 2 * d), lambda j: (j, 0)),
        out_shape=jax.ShapeDtypeStruct((grid * half, 2 * d), jnp.float32),
    )(tt)


def _tc_retile_out(flat, n_l, n_b, d, scale):
    inv = flat.reshape(n_l, n_b // 128, d, 128)
    nbb = n_b // 128
    bbq = 8

    def body(in_ref, out_ref):
        x = in_ref[0]
        cc = lax.broadcasted_iota(jnp.int32, (d, 2 * d), 1)
        pp = lax.broadcasted_iota(jnp.int32, (d, 2 * d), 0)
        s0 = (cc == 2 * pp).astype(jnp.bfloat16)
        s1 = (cc == 2 * pp + 1).astype(jnp.bfloat16)
        dn = (((1,), (0,)), ((), ()))
        xb = x.astype(jnp.bfloat16)
        y = lax.dot_general(
            xb[:, :, :d], s0, dn, preferred_element_type=jnp.float32
        ) + lax.dot_general(
            xb[:, :, d:], s1, dn, preferred_element_type=jnp.float32
        )
        y = y * scale
        for kk in range(d // 8):
            out_ref[0, kk] = y[:, kk * 8 : kk * 8 + 8, :]

    return pl.pallas_call(
        body,
        grid=(n_l, nbb // bbq),
        in_specs=[pl.BlockSpec((1, bbq, d, 128), lambda l, q: (l, q, 0, 0))],
        out_specs=pl.BlockSpec(
            (1, d // 8, bbq, 8, 128), lambda l, q: (l, 0, q, 0, 0)
        ),
        out_shape=jax.ShapeDtypeStruct((n_l, d // 8, nbb, 8, 128), jnp.float32),
    )(inv)


def _make_embed_kernel(n_total, d_model, n_per_w, chunk, num_cores):
    n_chunks = n_per_w // chunk
    n_outer = n_chunks // 2
    mesh = plsc.VectorSubcoreMesh(core_axis_name="c", subcore_axis_name="s")

    scratch = (
        [pltpu.VMEM((chunk,), jnp.int32) for _ in range(2)]
        + [pltpu.VMEM((chunk, d_model), jnp.float32) for _ in range(2)]
        + [pltpu.SemaphoreType.DMA for _ in range(4)]
    )

    @functools.partial(
        pl.kernel,
        mesh=mesh,
        out_type=jax.ShapeDtypeStruct((n_total, d_model), jnp.float32),
        compiler_params=pltpu.CompilerParams(use_tc_tiling_on_sc=False),
        scratch_types=scratch,
    )
    def k(idx_hbm, table_hbm, out_hbm, *rest):
        idxb = rest[0:2]
        rows = rest[2:4]
        isem = rest[4:6]
        gsem = rest[6:8]
        wid = lax.axis_index("s") * num_cores + lax.axis_index("c")
        base = wid * n_per_w

        def i_start(c, b):
            pltpu.make_async_copy(
                idx_hbm.at[pl.ds(base + c * chunk, chunk)], idxb[b], isem[b]
            ).start()

        def i_wait(c, b):
            pltpu.make_async_copy(
                idx_hbm.at[pl.ds(base + c * chunk, chunk)], idxb[b], isem[b]
            ).wait()

        def remap(b):
            for g in range(chunk // 16):
                sl = pl.ds(g * 16, 16)
                vv = idxb[b][sl]
                idxb[b][sl] = (
                    (vv & jnp.int32(-16384))
                    + ((vv & 8191) * 2)
                    + ((vv >> 13) & 1)
                )

        def g_start(b):
            pltpu.make_async_copy(table_hbm.at[idxb[b]], rows[b], gsem[b]).start()

        def g_wait(b):
            pltpu.make_async_copy(table_hbm.at[idxb[b]], rows[b], gsem[b]).wait()

        def o_sync(c, b):
            pltpu.sync_copy(rows[b], out_hbm.at[pl.ds(base + c * chunk, chunk)])

        i_start(0, 0)
        i_start(1, 1)
        i_wait(0, 0)
        remap(0)
        g_start(0)

        def outer(i2, _):
            for b in range(2):
                c = i2 * 2 + b
                g_wait(b)

                @pl.when(i2 < n_outer - 1)
                def _():
                    i_start(c + 2, b)
                    i_wait(c + 1, 1 - b)
                    remap(1 - b)
                    g_start(1 - b)

                if b == 0:

                    @pl.when(i2 == n_outer - 1)
                    def _():
                        i_wait(c + 1, 1 - b)
                        remap(1 - b)
                        g_start(1 - b)

                o_sync(c, b)
            return 0

        lax.fori_loop(0, n_outer, outer, 0)

    return k


def kernel(x, table):
    b, l = x.shape
    v, d = table.shape
    n_total = b * l
    idx_lmajor = jnp.transpose(x).reshape(n_total).astype(jnp.int32)
    packed = _tc_pack_table(table)
    tbl_rows = packed.reshape(packed.shape[0] * 2, d)
    info = plsc.get_sparse_core_info()
    nw = info.num_cores * info.num_subcores
    n_per_w = n_total // nw
    k = _make_embed_kernel(n_total, d, n_per_w, 256, info.num_cores)
    out_flat = k(idx_lmajor, tbl_rows)
    tiled = _tc_retile_out(
        out_flat.reshape(n_total * d), l, b, d, float(math.sqrt(d))
    )
    final = jnp.transpose(tiled, (2, 4, 0, 1, 3)).reshape(b, l, d)
    return final
